# baseline (speedup 1.0000x reference)
import functools

import jax
import jax.numpy as jnp
from jax import lax
from jax.experimental import pallas as pl
from jax.experimental.pallas import tpu as pltpu
from jax.experimental.pallas import tpu_sc as plsc

_NC = 2
_NS = 16
_NW = _NC * _NS
_CHUNK = 128


def _gather(idx, table):
    (b,) = idx.shape
    _, d = table.shape
    bpw = b // _NW
    cpw = bpw // _CHUNK
    mesh = plsc.VectorSubcoreMesh(core_axis_name="c", subcore_axis_name="s")

    @functools.partial(
        pl.kernel,
        mesh=mesh,
        out_type=jax.ShapeDtypeStruct((b, d), table.dtype),
        compiler_params=pltpu.CompilerParams(use_tc_tiling_on_sc=False),
        scratch_types=[
            pltpu.VMEM((bpw,), jnp.int32),
            pltpu.VMEM((bpw, d), table.dtype),
            pltpu.SemaphoreType.DMA,
        ],
    )
    def k(idx_hbm, table_hbm, out_hbm, idx_v, rows_v, sem):
        wid = lax.axis_index("s") * _NC + lax.axis_index("c")
        base = wid * bpw
        pltpu.sync_copy(idx_hbm.at[pl.ds(base, bpw)], idx_v)
        copies = [
            pltpu.async_copy(
                table_hbm.at[idx_v.at[pl.ds(j * _CHUNK, _CHUNK)]],
                rows_v.at[pl.ds(j * _CHUNK, _CHUNK)],
                sem,
            )
            for j in range(cpw)
        ]
        for c in copies:
            c.wait()
        pltpu.sync_copy(rows_v, out_hbm.at[pl.ds(base, bpw)])

    return k(idx, table)


def kernel(input, use_blank, has_blank, table):
    return _gather(input.astype(jnp.int32), table)


# speedup vs baseline: 2.3013x; 2.3013x over previous
import functools

import jax
import jax.numpy as jnp
from jax import lax
from jax.experimental import pallas as pl
from jax.experimental.pallas import tpu as pltpu
from jax.experimental.pallas import tpu_sc as plsc

_NC = 2
_NS = 16
_NW = _NC * _NS
_NB = 32
_L = 16


def _gather(idx, table3):
    (b,) = idx.shape
    _, r8, d = table3.shape
    bpw = b // _NW
    nbat = bpw // _NB
    mesh = plsc.VectorSubcoreMesh(core_axis_name="c", subcore_axis_name="s")

    @functools.partial(
        pl.kernel,
        mesh=mesh,
        out_type=jax.ShapeDtypeStruct((b, d), table3.dtype),
        scratch_types=[
            pltpu.VMEM((bpw + _L,), jnp.int32),
            pltpu.VMEM((2, _NB, r8, d), table3.dtype),
            pltpu.VMEM((2, _NB, d), table3.dtype),
            pltpu.SemaphoreType.DMA,
            pltpu.SemaphoreType.DMA,
            pltpu.SemaphoreType.DMA,
        ],
    )
    def k(idx_hbm, tbl_hbm, out_hbm, idx_v, tiles_v, rows_v,
          sem_g0, sem_g1, sem_o):
        wid = lax.axis_index("s") * _NC + lax.axis_index("c")
        base = wid * bpw
        pltpu.sync_copy(idx_hbm.at[pl.ds(base, bpw)], idx_v.at[pl.ds(0, bpw)])
        sems = (sem_g0, sem_g1)

        def issue_batch(g, buf):
            sem = sems[g % 2]

            def issue(i, _):
                vv = idx_v[pl.ds(g * _NB + i, _L)]
                t = lax.shift_right_logical(vv[0], 3)
                pltpu.async_copy(tbl_hbm.at[t], tiles_v.at[buf, i], sem)
                return ()

            lax.fori_loop(0, _NB, issue, ())

        def wait_batch(g, buf):
            sem = sems[g % 2]

            def wait1(i, _):
                pltpu.make_async_copy(
                    tbl_hbm.at[0], tiles_v.at[buf, i], sem).wait()
                return ()

            lax.fori_loop(0, _NB, wait1, ())

        issue_batch(0, 0)
        for g in range(nbat):
            p = g % 2
            if g + 1 < nbat:
                issue_batch(g + 1, 1 - p)
            wait_batch(g, p)
            if g >= 2:
                pltpu.make_async_copy(
                    rows_v.at[p], out_hbm.at[pl.ds(base + (g - 2) * _NB, _NB)],
                    sem_o).wait()

            def extract(i, _):
                vv = idx_v[pl.ds(g * _NB + i, _L)]
                r = jnp.bitwise_and(vv[0], 7)
                for c in range(d // _L):
                    rows_v[p, i, pl.ds(c * _L, _L)] = (
                        tiles_v[p, i, r, pl.ds(c * _L, _L)])
                return ()

            lax.fori_loop(0, _NB, extract, ())
            pltpu.async_copy(
                rows_v.at[p], out_hbm.at[pl.ds(base + g * _NB, _NB)], sem_o)
        for g in range(nbat - 2, nbat):
            p = g % 2
            pltpu.make_async_copy(
                rows_v.at[p], out_hbm.at[pl.ds(base + g * _NB, _NB)],
                sem_o).wait()

    return k(idx, table3)


def kernel(input, use_blank, has_blank, table):
    v, d = table.shape
    table3 = table.reshape(v // 8, 8, d)
    return _gather(input.astype(jnp.int32), table3)
